# baseline (speedup 1.0000x reference)
import functools
import jax
import jax.numpy as jnp
from jax import lax
from jax.experimental import pallas as pl
from jax.experimental.pallas import tpu as pltpu, tpu_sc as plsc

NC = 2
NS = 16
NW = NC * NS
C = 32
L = 16
U = 16


def kernel(x, emb):
    B, S, D = x.shape
    xf = x.reshape(B * S * D)
    embf = emb.reshape(S * D)
    seq_per_w = S // NW
    n_chunks = seq_per_w // C
    W = C * D
    n_vec = W // (L * U)

    mesh = plsc.VectorSubcoreMesh(core_axis_name="c", subcore_axis_name="s")

    @functools.partial(
        pl.kernel,
        out_type=jax.ShapeDtypeStruct((B * S * D,), jnp.float32),
        mesh=mesh,
        scratch_types=[
            pltpu.VMEM((W,), jnp.float32),
            pltpu.VMEM((W,), jnp.float32),
        ],
    )
    def sc_add(xf_hbm, emb_hbm, out_hbm, buf, emb_buf):
        cid = lax.axis_index("c")
        sid = lax.axis_index("s")
        wid = cid * NS + sid

        def chunk_body(i, carry):
            seq_base = (wid * seq_per_w + i * C) * D
            pltpu.sync_copy(emb_hbm.at[pl.ds(seq_base, W)], emb_buf)

            def batch_body(b, carry2):
                base = b * S * D + seq_base
                pltpu.sync_copy(xf_hbm.at[pl.ds(base, W)], buf)

                def add_body(v, carry3):
                    k0 = v * (L * U)
                    for u in range(U):
                        k = k0 + u * L
                        plsc.addupdate(buf.at[pl.ds(k, L)], emb_buf[pl.ds(k, L)])
                    return carry3

                lax.fori_loop(0, n_vec, add_body, 0)
                pltpu.sync_copy(buf, out_hbm.at[pl.ds(base, W)])
                return carry2

            lax.fori_loop(0, B, batch_body, 0)
            return carry

        lax.fori_loop(0, n_chunks, chunk_body, 0)

    return sc_add(xf, embf).reshape(B, S, D)


# speedup vs baseline: 1.1774x; 1.1774x over previous
import functools
import jax
import jax.numpy as jnp
from jax import lax
from jax.experimental import pallas as pl
from jax.experimental.pallas import tpu as pltpu, tpu_sc as plsc

NC = 2
NS = 16
NW = NC * NS
C = 16
L = 16
U = 16
NB = 3


def kernel(x, emb):
    B, S, D = x.shape
    xf = x.reshape(B * S * D)
    embf = emb.reshape(S * D)
    seq_per_w = S // NW
    n_chunks = seq_per_w // C
    W = C * D
    n_vec = W // (L * U)
    T = n_chunks * B

    mesh = plsc.VectorSubcoreMesh(core_axis_name="c", subcore_axis_name="s")

    @functools.partial(
        pl.kernel,
        out_type=jax.ShapeDtypeStruct((B * S * D,), jnp.float32),
        mesh=mesh,
        scratch_types=[
            [pltpu.VMEM((W,), jnp.float32)] * NB,
            [pltpu.VMEM((W,), jnp.float32)] * 2,
            [pltpu.SemaphoreType.DMA] * NB,
            [pltpu.SemaphoreType.DMA] * NB,
            [pltpu.SemaphoreType.DMA] * 2,
        ],
    )
    def sc_add(xf_hbm, emb_hbm, out_hbm, xbufs, ebufs, sin, sout, semb):
        cid = lax.axis_index("c")
        sid = lax.axis_index("s")
        wid = cid * NS + sid
        w_base = wid * seq_per_w * D

        def x_base(t):
            return (t % B) * S * D + w_base + (t // B) * W

        def start_in(t):
            return pltpu.async_copy(
                xf_hbm.at[pl.ds(x_base(t), W)], xbufs[t % NB], sin[t % NB]
            )

        def start_emb(c):
            return pltpu.async_copy(
                emb_hbm.at[pl.ds(w_base + c * W, W)], ebufs[c % 2], semb[c % 2]
            )

        in_d, out_d, emb_d = {}, {}, {}
        emb_d[0] = start_emb(0)
        in_d[0] = start_in(0)
        in_d[1] = start_in(1)

        for t in range(T):
            p = t % NB
            c = t // B
            if t % B == 0:
                emb_d[c].wait()
                if c + 1 < n_chunks:
                    emb_d[c + 1] = start_emb(c + 1)
            in_d[t].wait()
            if t + 2 < T:
                if t >= 1:
                    out_d[t - 1].wait()
                in_d[t + 2] = start_in(t + 2)

            buf, ebuf = xbufs[p], ebufs[c % 2]

            def add_body(v, carry):
                k0 = v * (L * U)
                for u in range(U):
                    k = k0 + u * L
                    plsc.addupdate(buf.at[pl.ds(k, L)], ebuf[pl.ds(k, L)])
                return carry

            lax.fori_loop(0, n_vec, add_body, 0)
            out_d[t] = pltpu.async_copy(
                buf, out_hbm.at[pl.ds(x_base(t), W)], sout[p]
            )

        for t in range(max(T - 3, 0), T):
            out_d[t].wait()

    return sc_add(xf, embf).reshape(B, S, D)


# speedup vs baseline: 5.1067x; 4.3374x over previous
import jax
import jax.numpy as jnp
from jax.experimental import pallas as pl
from jax.experimental.pallas import tpu as pltpu

BS = 8192
BD = 256


def _add_kernel(x_ref, emb_ref, o_ref):
    o_ref[...] = x_ref[...] + emb_ref[...][None]


def kernel(x, emb):
    B, S, D = x.shape
    grid = (S // BS, D // BD, B)
    return pl.pallas_call(
        _add_kernel,
        grid=grid,
        in_specs=[
            pl.BlockSpec((1, BS, BD), lambda s, d, b: (b, s, d)),
            pl.BlockSpec((BS, BD), lambda s, d, b: (s, d)),
        ],
        out_specs=pl.BlockSpec((1, BS, BD), lambda s, d, b: (b, s, d)),
        out_shape=jax.ShapeDtypeStruct((B, S, D), x.dtype),
        compiler_params=pltpu.CompilerParams(
            dimension_semantics=("arbitrary", "arbitrary", "arbitrary"),
        ),
    )(x, emb)


# speedup vs baseline: 5.1262x; 1.0038x over previous
import jax
import jax.numpy as jnp
from jax.experimental import pallas as pl
from jax.experimental.pallas import tpu as pltpu

BS = 4096
BD = 512


def _add_kernel(x_ref, emb_ref, o_ref):
    o_ref[...] = x_ref[...] + emb_ref[...][None]


def kernel(x, emb):
    B, S, D = x.shape
    grid = (S // BS, D // BD, B)
    return pl.pallas_call(
        _add_kernel,
        grid=grid,
        in_specs=[
            pl.BlockSpec((1, BS, BD), lambda s, d, b: (b, s, d)),
            pl.BlockSpec((BS, BD), lambda s, d, b: (s, d)),
        ],
        out_specs=pl.BlockSpec((1, BS, BD), lambda s, d, b: (b, s, d)),
        out_shape=jax.ShapeDtypeStruct((B, S, D), x.dtype),
        compiler_params=pltpu.CompilerParams(
            dimension_semantics=("arbitrary", "arbitrary", "arbitrary"),
        ),
    )(x, emb)


# speedup vs baseline: 5.1689x; 1.0083x over previous
import jax
import jax.numpy as jnp
from jax.experimental import pallas as pl
from jax.experimental.pallas import tpu as pltpu

BS = 2048


def _add_kernel(x_ref, emb_ref, o_ref):
    o_ref[...] = x_ref[...] + emb_ref[...][None]


def kernel(x, emb):
    B, S, D = x.shape
    grid = (S // BS, B)
    return pl.pallas_call(
        _add_kernel,
        grid=grid,
        in_specs=[
            pl.BlockSpec((1, BS, D), lambda s, b: (b, s, 0)),
            pl.BlockSpec((BS, D), lambda s, b: (s, 0)),
        ],
        out_specs=pl.BlockSpec((1, BS, D), lambda s, b: (b, s, 0)),
        out_shape=jax.ShapeDtypeStruct((B, S, D), x.dtype),
        compiler_params=pltpu.CompilerParams(
            dimension_semantics=("parallel", "arbitrary"),
        ),
    )(x, emb)
